# Initial kernel scaffold; baseline (speedup 1.0000x reference)
import jax
import jax.numpy as jnp
from jax.experimental import pallas as pl


def kernel(hidden, W_router, W1, b1, W2, b2):
    raise NotImplementedError("write your pallas kernel here")


# speedup vs baseline: 1.9593x; 1.9593x over previous
import functools

import jax
import jax.numpy as jnp
from jax import lax
from jax.experimental import pallas as pl
from jax.experimental.pallas import tpu as pltpu
from jax.experimental.pallas import tpu_sc as plsc

E = 8
TOP_K = 2
D = 768
DFF = 2048
T = 2048
CAP = 1024

BLK = 256
BLK_SHIFT = 8
NB = (T * TOP_K + E * (BLK - 1)) // BLK
NR = NB * BLK
XS_ROWS = NR + BLK
DUMP = NR

NC = 2
NS = 16
NW = NC * NS
PAIRS = T * TOP_K
PPT = PAIRS // NW
TPT = T // NW
VPT = PPT // 16


def _lane():
    return lax.broadcasted_iota(jnp.int32, (16,), 0)


def _router_body(x_ref, wr_ref, ids_ref, pr_ref, cnt_ref):
    logits = jnp.dot(x_ref[...], wr_ref[...],
                     preferred_element_type=jnp.float32)
    idx8 = lax.broadcasted_iota(jnp.int32, (T, E), 1)
    v1 = logits[:, 0:1]
    a1 = jnp.zeros((T, 1), jnp.int32)
    for e in range(1, E):
        le = logits[:, e:e + 1]
        upd = le > v1
        a1 = jnp.where(upd, e, a1)
        v1 = jnp.where(upd, le, v1)
    v2 = jnp.full((T, 1), -jnp.inf, jnp.float32)
    a2 = jnp.full((T, 1), E, jnp.int32)
    for e in range(E):
        le = logits[:, e:e + 1]
        upd = (e != a1) & (le > v2)
        a2 = jnp.where(upd, e, a2)
        v2 = jnp.where(upd, le, v2)
    p1 = 1.0 / (1.0 + jnp.exp(v2 - v1))
    ids_ref[:, 0:1] = a1
    ids_ref[:, 1:2] = a2
    pr_ref[:, 0:1] = p1
    pr_ref[:, 1:2] = 1.0 - p1
    onehot = (idx8 == a1).astype(jnp.int32) + (idx8 == a2).astype(jnp.int32)
    cnt_ref[...] = jnp.sum(onehot.reshape(NW, T // NW, E), axis=1)


def _lane_scalar(vec, lane):
    return jnp.sum(jnp.where(_lane() == lane, vec, jnp.zeros_like(vec)))


def _dispatch_body(ids_hbm, pr_hbm, cnt_hbm, x_hbm,
                   xs_hbm, dest_hbm, gate_hbm, meta_hbm,
                   ids_v, pr_v, cnt_v, dest_v, gate_v, tok_v, meta_v,
                   fold_v, rows_v, sem):
    LANE = _lane()
    wid = lax.axis_index("s") * NC + lax.axis_index("c")
    base = wid * PPT

    pltpu.sync_copy(ids_hbm.at[pl.ds(base, PPT)], ids_v)
    pltpu.sync_copy(pr_hbm.at[pl.ds(base, PPT)], pr_v)
    pltpu.sync_copy(cnt_hbm, cnt_v)

    zero = jnp.zeros((16,), jnp.int32)
    tot = zero
    mybase = zero
    lo_half = LANE < 8
    for i in range(NW // 2):
        v = cnt_v[pl.ds(16 * i, 16)]
        chunk = jnp.where(lo_half, 2 * i, 2 * i + 1)
        tot = tot + v
        mybase = mybase + jnp.where(chunk < wid, v, zero)
    def fold(vec):
        fold_v[pl.ds(0, 16)] = vec
        hi = fold_v[pl.ds(8, 16)]
        return jnp.where(lo_half, vec + hi, zero)
    tot = fold(tot)
    mybase = fold(mybase)

    kept = jnp.minimum(tot, CAP)
    padded = ((kept + (BLK - 1)) >> BLK_SHIFT) << BLK_SHIFT
    padded = jnp.where(lo_half, padded, zero)
    off = plsc.cumsum(padded) - padded
    used_rows = jnp.sum(padded)
    used_blocks = used_rows >> BLK_SHIFT

    base_e = [_lane_scalar(mybase, e) for e in range(E)]
    off_e = [_lane_scalar(off, e) for e in range(E)]
    pad_e = [_lane_scalar(padded, e) for e in range(E)]

    @pl.when(wid == 0)
    def _write_meta():
        for c in range(2):
            lane_g = LANE + 16 * c
            bidx = jnp.clip(lane_g - 1, 0, used_blocks - 1)
            bb = bidx << BLK_SHIFT
            bev = jnp.zeros((16,), jnp.int32)
            for e in range(E):
                hit = (bb >= off_e[e]) & (bb < off_e[e] + pad_e[e])
                bev = jnp.where(hit, e, bev)
            if c == 0:
                bev = jnp.where(lane_g == 0, used_blocks, bev)
            meta_v[pl.ds(16 * c, 16)] = bev
        pltpu.sync_copy(meta_v, meta_hbm)

    run = [jnp.int32(0)] * E
    for j in range(VPT):
        ev = ids_v[pl.ds(16 * j, 16)]
        gv = pr_v[pl.ds(16 * j, 16)]
        dest = jnp.full((16,), DUMP, jnp.int32)
        for e in range(E):
            m = ev == e
            mi = m.astype(jnp.int32)
            incl = plsc.cumsum(mi)
            pos = base_e[e] + run[e] + incl - 1
            ok = m & (pos < CAP)
            dest = jnp.where(ok, off_e[e] + pos, dest)
            gv = jnp.where(m & (pos >= CAP), jnp.float32(0.0), gv)
            run[e] = run[e] + jnp.sum(mi)
        dest_v[pl.ds(16 * j, 16)] = dest
        gate_v[pl.ds(16 * j, 16)] = gv
        tok_v[pl.ds(16 * j, 16)] = (LANE + (base + 16 * j)) >> 1

    pltpu.sync_copy(dest_v, dest_hbm.at[pl.ds(base, PPT)])
    pltpu.sync_copy(gate_v, gate_hbm.at[pl.ds(base, PPT)])

    pltpu.async_copy(x_hbm.at[tok_v], rows_v, sem).wait()
    pltpu.sync_copy(rows_v, xs_hbm.at[dest_v])


def _mlp_body(meta_ref, xs_ref, w1_ref, b1_ref, w2_ref, b2_ref, out_ref):
    b = pl.program_id(0)

    @pl.when(b < meta_ref[0])
    def _compute():
        xb = xs_ref[...].astype(jnp.bfloat16)
        h = jnp.dot(xb, w1_ref[0], preferred_element_type=jnp.float32)
        h = jnp.maximum(h + b1_ref[0], 0.0).astype(jnp.bfloat16)
        out = jnp.dot(h, w2_ref[0], preferred_element_type=jnp.float32)
        out_ref[...] = out + b2_ref[0]


def _combine_body(rows_hbm, dest_hbm, gate_hbm, y_hbm,
                  d0_v, d1_v, gate_v, rows_v, out_v, sem):
    LANE = _lane()
    wid = lax.axis_index("s") * NC + lax.axis_index("c")
    base = wid * PPT
    pltpu.sync_copy(gate_hbm.at[pl.ds(base, PPT)], gate_v)
    half_tok = TPT // 2
    for h in range(2):
        dv = d0_v if h == 0 else d1_v
        pltpu.sync_copy(dest_hbm.at[pl.ds(base + 64 * h, 64)], dv)
        pltpu.async_copy(rows_hbm.at[dv], rows_v, sem).wait()

        def body(tt, _):
            p0 = 64 * h + 2 * tt
            g0 = plsc.load_gather(gate_v, [jnp.full((16,), p0, jnp.int32)])
            g1 = plsc.load_gather(gate_v, [jnp.full((16,), p0 + 1, jnp.int32)])
            r0row = jnp.full((16,), 2 * tt, jnp.int32)
            r1row = jnp.full((16,), 2 * tt + 1, jnp.int32)
            orow = jnp.full((16,), tt, jnp.int32)
            fzero = jnp.zeros((16,), jnp.float32)
            for c in range(D // 16):
                col = LANE + 16 * c
                r0 = plsc.load_gather(rows_v, [r0row, col])
                r1 = plsc.load_gather(rows_v, [r1row, col])
                acc = (jnp.where(g0 != 0.0, g0 * r0, fzero)
                       + jnp.where(g1 != 0.0, g1 * r1, fzero))
                plsc.store_scatter(out_v, [orow, col], acc)
            return 0

        lax.fori_loop(0, half_tok, body, 0)
        pltpu.sync_copy(out_v, y_hbm.at[pl.ds(wid * TPT + half_tok * h,
                                              half_tok)])


@functools.partial(jax.jit, static_argnames=())
def kernel(hidden, W_router, W1, b1, W2, b2):
    x = hidden.reshape(T, D)

    ids, probs, cnt = pl.pallas_call(
        _router_body,
        out_shape=(
            jax.ShapeDtypeStruct((T, TOP_K), jnp.int32),
            jax.ShapeDtypeStruct((T, TOP_K), jnp.float32),
            jax.ShapeDtypeStruct((NW, E), jnp.int32),
        ),
    )(x, W_router)

    ids_flat = ids.reshape(PAIRS)
    pr_flat = probs.reshape(PAIRS)
    cnt_flat = cnt.reshape(NW * E)

    sc_mesh = plsc.VectorSubcoreMesh(core_axis_name="c", subcore_axis_name="s",
                                     num_cores=NC, num_subcores=NS)

    xs, dest, gate, meta = pl.kernel(
        _dispatch_body,
        out_type=(
            jax.ShapeDtypeStruct((XS_ROWS, D), jnp.float32),
            jax.ShapeDtypeStruct((PAIRS,), jnp.int32),
            jax.ShapeDtypeStruct((PAIRS,), jnp.float32),
            jax.ShapeDtypeStruct((32,), jnp.int32),
        ),
        mesh=sc_mesh,
        compiler_params=pltpu.CompilerParams(needs_layout_passes=False),
        scratch_types=[
            pltpu.VMEM((PPT,), jnp.int32),
            pltpu.VMEM((PPT,), jnp.float32),
            pltpu.VMEM((NW * E,), jnp.int32),
            pltpu.VMEM((PPT,), jnp.int32),
            pltpu.VMEM((PPT,), jnp.float32),
            pltpu.VMEM((PPT,), jnp.int32),
            pltpu.VMEM((32,), jnp.int32),
            pltpu.VMEM((24,), jnp.int32),
            pltpu.VMEM((PPT, D), jnp.float32),
            pltpu.SemaphoreType.DMA,
        ],
    )(ids_flat, pr_flat, cnt_flat, x)

    w1c = W1.astype(jnp.bfloat16)
    w2c = W2.astype(jnp.bfloat16)
    b1r = b1.reshape(E, 1, DFF)
    b2r = b2.reshape(E, 1, D)

    out_rows = pl.pallas_call(
        _mlp_body,
        grid_spec=pltpu.PrefetchScalarGridSpec(
            num_scalar_prefetch=1,
            grid=(NB,),
            in_specs=[
                pl.BlockSpec((BLK, D), lambda b, m: (b, 0)),
                pl.BlockSpec((1, D, DFF), lambda b, m: (m[1 + b], 0, 0)),
                pl.BlockSpec((1, 1, DFF), lambda b, m: (m[1 + b], 0, 0)),
                pl.BlockSpec((1, DFF, D), lambda b, m: (m[1 + b], 0, 0)),
                pl.BlockSpec((1, 1, D), lambda b, m: (m[1 + b], 0, 0)),
            ],
            out_specs=pl.BlockSpec((BLK, D), lambda b, m: (b, 0)),
        ),
        out_shape=jax.ShapeDtypeStruct((XS_ROWS, D), jnp.float32),
    )(meta, xs, w1c, b1r, w2c, b2r)

    y = pl.kernel(
        _combine_body,
        out_type=jax.ShapeDtypeStruct((T, D), jnp.float32),
        mesh=sc_mesh,
        compiler_params=pltpu.CompilerParams(needs_layout_passes=False),
        scratch_types=[
            pltpu.VMEM((64,), jnp.int32),
            pltpu.VMEM((64,), jnp.int32),
            pltpu.VMEM((PPT,), jnp.float32),
            pltpu.VMEM((64, D), jnp.float32),
            pltpu.VMEM((TPT // 2, D), jnp.float32),
            pltpu.SemaphoreType.DMA,
        ],
    )(out_rows, dest, gate)

    return y.reshape(hidden.shape)


# speedup vs baseline: 2.3177x; 1.1830x over previous
import functools

import jax
import jax.numpy as jnp
from jax import lax
from jax.experimental import pallas as pl
from jax.experimental.pallas import tpu as pltpu
from jax.experimental.pallas import tpu_sc as plsc

E = 8
TOP_K = 2
D = 768
DFF = 2048
T = 2048
CAP = 1024

BLK = 256
BLK_SHIFT = 8
NB = (T * TOP_K + E * (BLK - 1)) // BLK
NR = NB * BLK
XS_ROWS = NR + BLK
DUMP = NR

NC = 2
NS = 16
NW = NC * NS
PAIRS = T * TOP_K
PPT = PAIRS // NW
TPT = T // NW
VPT = PPT // 16


def _lane():
    return lax.broadcasted_iota(jnp.int32, (16,), 0)


def _router_body(x_ref, wr_ref, ids_ref, pr_ref, cnt_ref):
    logits = jnp.dot(x_ref[...], wr_ref[...],
                     preferred_element_type=jnp.float32)
    idx8 = lax.broadcasted_iota(jnp.int32, (T, E), 1)
    v1 = logits[:, 0:1]
    a1 = jnp.zeros((T, 1), jnp.int32)
    for e in range(1, E):
        le = logits[:, e:e + 1]
        upd = le > v1
        a1 = jnp.where(upd, e, a1)
        v1 = jnp.where(upd, le, v1)
    v2 = jnp.full((T, 1), -jnp.inf, jnp.float32)
    a2 = jnp.full((T, 1), E, jnp.int32)
    for e in range(E):
        le = logits[:, e:e + 1]
        upd = (e != a1) & (le > v2)
        a2 = jnp.where(upd, e, a2)
        v2 = jnp.where(upd, le, v2)
    p1 = 1.0 / (1.0 + jnp.exp(v2 - v1))
    ids_ref[:, 0:1] = a1
    ids_ref[:, 1:2] = a2
    pr_ref[:, 0:1] = p1
    pr_ref[:, 1:2] = 1.0 - p1
    onehot = (idx8 == a1).astype(jnp.int32) + (idx8 == a2).astype(jnp.int32)
    cnt_ref[...] = jnp.sum(onehot.reshape(NW, T // NW, E), axis=1)


def _lane_scalar(vec, lane):
    return jnp.sum(jnp.where(_lane() == lane, vec, jnp.zeros_like(vec)))


def _dispatch_body(ids_hbm, pr_hbm, cnt_hbm, x_hbm,
                   xs_hbm, dest_hbm, gate_hbm, meta_hbm,
                   ids_v, pr_v, cnt_v, dest_v, gate_v, tok_v, meta_v,
                   fold_v, rows_v, sem):
    LANE = _lane()
    wid = lax.axis_index("s") * NC + lax.axis_index("c")
    base = wid * PPT

    pltpu.sync_copy(ids_hbm.at[pl.ds(base, PPT)], ids_v)
    pltpu.sync_copy(pr_hbm.at[pl.ds(base, PPT)], pr_v)
    pltpu.sync_copy(cnt_hbm, cnt_v)

    zero = jnp.zeros((16,), jnp.int32)
    tot = zero
    mybase = zero
    lo_half = LANE < 8
    for i in range(NW // 2):
        v = cnt_v[pl.ds(16 * i, 16)]
        chunk = jnp.where(lo_half, 2 * i, 2 * i + 1)
        tot = tot + v
        mybase = mybase + jnp.where(chunk < wid, v, zero)
    def fold(vec):
        fold_v[pl.ds(0, 16)] = vec
        hi = fold_v[pl.ds(8, 16)]
        return jnp.where(lo_half, vec + hi, zero)
    tot = fold(tot)
    mybase = fold(mybase)

    kept = jnp.minimum(tot, CAP)
    padded = ((kept + (BLK - 1)) >> BLK_SHIFT) << BLK_SHIFT
    padded = jnp.where(lo_half, padded, zero)
    off = plsc.cumsum(padded) - padded
    used_rows = jnp.sum(padded)
    used_blocks = used_rows >> BLK_SHIFT

    base_e = [_lane_scalar(mybase, e) for e in range(E)]
    off_e = [_lane_scalar(off, e) for e in range(E)]
    pad_e = [_lane_scalar(padded, e) for e in range(E)]

    @pl.when(wid == 0)
    def _write_meta():
        for c in range(2):
            lane_g = LANE + 16 * c
            bidx = jnp.clip(lane_g - 1, 0, used_blocks - 1)
            bb = bidx << BLK_SHIFT
            bev = jnp.zeros((16,), jnp.int32)
            for e in range(E):
                hit = (bb >= off_e[e]) & (bb < off_e[e] + pad_e[e])
                bev = jnp.where(hit, e, bev)
            if c == 0:
                bev = jnp.where(lane_g == 0, used_blocks, bev)
            meta_v[pl.ds(16 * c, 16)] = bev
        pltpu.sync_copy(meta_v, meta_hbm)

    run = [jnp.int32(0)] * E
    for j in range(VPT):
        ev = ids_v[pl.ds(16 * j, 16)]
        gv = pr_v[pl.ds(16 * j, 16)]
        dest = jnp.full((16,), DUMP, jnp.int32)
        for e in range(E):
            m = ev == e
            mi = m.astype(jnp.int32)
            incl = plsc.cumsum(mi)
            pos = base_e[e] + run[e] + incl - 1
            ok = m & (pos < CAP)
            dest = jnp.where(ok, off_e[e] + pos, dest)
            gv = jnp.where(m & (pos >= CAP), jnp.float32(0.0), gv)
            run[e] = run[e] + jnp.sum(mi)
        dest_v[pl.ds(16 * j, 16)] = dest
        gate_v[pl.ds(16 * j, 16)] = gv
        tok_v[pl.ds(16 * j, 16)] = (LANE + (base + 16 * j)) >> 1

    pltpu.sync_copy(dest_v, dest_hbm.at[pl.ds(base, PPT)])
    pltpu.sync_copy(gate_v, gate_hbm.at[pl.ds(base, PPT)])

    pltpu.async_copy(x_hbm.at[tok_v], rows_v, sem).wait()
    pltpu.sync_copy(rows_v, xs_hbm.at[dest_v])


def _mlp_body(meta_ref, xs_ref, w1_ref, b1_ref, w2_ref, b2_ref, out_ref):
    b = pl.program_id(0)

    @pl.when(b < meta_ref[0])
    def _compute():
        xb = xs_ref[...].astype(jnp.bfloat16)
        w1b = w1_ref[0].astype(jnp.bfloat16)
        h = jnp.dot(xb, w1b, preferred_element_type=jnp.float32)
        h = jnp.maximum(h + b1_ref[0], 0.0).astype(jnp.bfloat16)
        w2b = w2_ref[0].astype(jnp.bfloat16)
        out = jnp.dot(h, w2b, preferred_element_type=jnp.float32)
        out_ref[...] = out + b2_ref[0]


def _combine_body(rows_hbm, dest_hbm, gate_hbm, y_hbm,
                  d0_v, d1_v, gate_v, rows_v, out_v, sem):
    LANE = _lane()
    wid = lax.axis_index("s") * NC + lax.axis_index("c")
    base = wid * PPT
    pltpu.sync_copy(gate_hbm.at[pl.ds(base, PPT)], gate_v)
    half_tok = TPT // 2
    for h in range(2):
        dv = d0_v if h == 0 else d1_v
        pltpu.sync_copy(dest_hbm.at[pl.ds(base + 64 * h, 64)], dv)
        pltpu.async_copy(rows_hbm.at[dv], rows_v, sem).wait()

        def body(tt, _):
            p0 = 64 * h + 2 * tt
            g0 = plsc.load_gather(gate_v, [jnp.full((16,), p0, jnp.int32)])
            g1 = plsc.load_gather(gate_v, [jnp.full((16,), p0 + 1, jnp.int32)])
            r0row = jnp.full((16,), 2 * tt, jnp.int32)
            r1row = jnp.full((16,), 2 * tt + 1, jnp.int32)
            orow = jnp.full((16,), tt, jnp.int32)
            fzero = jnp.zeros((16,), jnp.float32)
            for c in range(D // 16):
                col = LANE + 16 * c
                r0 = plsc.load_gather(rows_v, [r0row, col])
                r1 = plsc.load_gather(rows_v, [r1row, col])
                acc = (jnp.where(g0 != 0.0, g0 * r0, fzero)
                       + jnp.where(g1 != 0.0, g1 * r1, fzero))
                plsc.store_scatter(out_v, [orow, col], acc)
            return 0

        lax.fori_loop(0, half_tok, body, 0)
        pltpu.sync_copy(out_v, y_hbm.at[pl.ds(wid * TPT + half_tok * h,
                                              half_tok)])


@functools.partial(jax.jit, static_argnames=())
def kernel(hidden, W_router, W1, b1, W2, b2):
    x = hidden.reshape(T, D)

    ids, probs, cnt = pl.pallas_call(
        _router_body,
        out_shape=(
            jax.ShapeDtypeStruct((T, TOP_K), jnp.int32),
            jax.ShapeDtypeStruct((T, TOP_K), jnp.float32),
            jax.ShapeDtypeStruct((NW, E), jnp.int32),
        ),
    )(x, W_router)

    ids_flat = ids.reshape(PAIRS)
    pr_flat = probs.reshape(PAIRS)
    cnt_flat = cnt.reshape(NW * E)

    sc_mesh = plsc.VectorSubcoreMesh(core_axis_name="c", subcore_axis_name="s",
                                     num_cores=NC, num_subcores=NS)

    xs, dest, gate, meta = pl.kernel(
        _dispatch_body,
        out_type=(
            jax.ShapeDtypeStruct((XS_ROWS, D), jnp.float32),
            jax.ShapeDtypeStruct((PAIRS,), jnp.int32),
            jax.ShapeDtypeStruct((PAIRS,), jnp.float32),
            jax.ShapeDtypeStruct((32,), jnp.int32),
        ),
        mesh=sc_mesh,
        compiler_params=pltpu.CompilerParams(needs_layout_passes=False),
        scratch_types=[
            pltpu.VMEM((PPT,), jnp.int32),
            pltpu.VMEM((PPT,), jnp.float32),
            pltpu.VMEM((NW * E,), jnp.int32),
            pltpu.VMEM((PPT,), jnp.int32),
            pltpu.VMEM((PPT,), jnp.float32),
            pltpu.VMEM((PPT,), jnp.int32),
            pltpu.VMEM((32,), jnp.int32),
            pltpu.VMEM((24,), jnp.int32),
            pltpu.VMEM((PPT, D), jnp.float32),
            pltpu.SemaphoreType.DMA,
        ],
    )(ids_flat, pr_flat, cnt_flat, x)

    b1r = b1.reshape(E, 1, DFF)
    b2r = b2.reshape(E, 1, D)

    out_rows = pl.pallas_call(
        _mlp_body,
        grid_spec=pltpu.PrefetchScalarGridSpec(
            num_scalar_prefetch=1,
            grid=(NB,),
            in_specs=[
                pl.BlockSpec((BLK, D), lambda b, m: (b, 0)),
                pl.BlockSpec((1, D, DFF), lambda b, m: (m[1 + b], 0, 0)),
                pl.BlockSpec((1, 1, DFF), lambda b, m: (m[1 + b], 0, 0)),
                pl.BlockSpec((1, DFF, D), lambda b, m: (m[1 + b], 0, 0)),
                pl.BlockSpec((1, 1, D), lambda b, m: (m[1 + b], 0, 0)),
            ],
            out_specs=pl.BlockSpec((BLK, D), lambda b, m: (b, 0)),
        ),
        out_shape=jax.ShapeDtypeStruct((XS_ROWS, D), jnp.float32),
    )(meta, xs, W1, b1r, W2, b2r)

    y = pl.kernel(
        _combine_body,
        out_type=jax.ShapeDtypeStruct((T, D), jnp.float32),
        mesh=sc_mesh,
        compiler_params=pltpu.CompilerParams(needs_layout_passes=False),
        scratch_types=[
            pltpu.VMEM((64,), jnp.int32),
            pltpu.VMEM((64,), jnp.int32),
            pltpu.VMEM((PPT,), jnp.float32),
            pltpu.VMEM((64, D), jnp.float32),
            pltpu.VMEM((TPT // 2, D), jnp.float32),
            pltpu.SemaphoreType.DMA,
        ],
    )(out_rows, dest, gate)

    return y.reshape(hidden.shape)


# speedup vs baseline: 2.3772x; 1.0257x over previous
import functools

import jax
import jax.numpy as jnp
from jax import lax
from jax.experimental import pallas as pl
from jax.experimental.pallas import tpu as pltpu
from jax.experimental.pallas import tpu_sc as plsc

E = 8
TOP_K = 2
D = 768
DFF = 2048
T = 2048
CAP = 1024

BLK = 256
BLK_SHIFT = 8
NB = (T * TOP_K + E * (BLK - 1)) // BLK
NR = NB * BLK
XS_ROWS = NR + BLK
DUMP = NR

NC = 2
NS = 16
NW = NC * NS
PAIRS = T * TOP_K
PPT = PAIRS // NW
TPT = T // NW
VPT = PPT // 16


def _lane():
    return lax.broadcasted_iota(jnp.int32, (16,), 0)


def _router_body(x_ref, wr_ref, ids_ref, pr_ref, cnt_ref):
    logits = jnp.dot(x_ref[...], wr_ref[...],
                     preferred_element_type=jnp.float32)
    idx8 = lax.broadcasted_iota(jnp.int32, (T, E), 1)
    v1 = logits[:, 0:1]
    a1 = jnp.zeros((T, 1), jnp.int32)
    for e in range(1, E):
        le = logits[:, e:e + 1]
        upd = le > v1
        a1 = jnp.where(upd, e, a1)
        v1 = jnp.where(upd, le, v1)
    v2 = jnp.full((T, 1), -jnp.inf, jnp.float32)
    a2 = jnp.full((T, 1), E, jnp.int32)
    for e in range(E):
        le = logits[:, e:e + 1]
        upd = (e != a1) & (le > v2)
        a2 = jnp.where(upd, e, a2)
        v2 = jnp.where(upd, le, v2)
    p1 = 1.0 / (1.0 + jnp.exp(v2 - v1))
    ids_ref[:, 0:1] = a1
    ids_ref[:, 1:2] = a2
    pr_ref[:, 0:1] = p1
    pr_ref[:, 1:2] = 1.0 - p1
    onehot = (idx8 == a1).astype(jnp.int32) + (idx8 == a2).astype(jnp.int32)
    cnt_ref[...] = jnp.sum(onehot.reshape(NW, T // NW, E), axis=1)


def _lane_scalar(vec, lane):
    return jnp.sum(jnp.where(_lane() == lane, vec, jnp.zeros_like(vec)))


def _dispatch_body(ids_hbm, pr_hbm, cnt_hbm, x_hbm,
                   xs_hbm, dest_hbm, gate_hbm, meta_hbm,
                   ids_v, pr_v, cnt_v, dest_v, gate_v, d0_v, d1_v, meta_v,
                   fold_v, rows_v, sem):
    LANE = _lane()
    wid = lax.axis_index("s") * NC + lax.axis_index("c")
    base = wid * PPT

    rows_dma = pltpu.async_copy(x_hbm.at[pl.ds(wid * TPT, TPT)], rows_v, sem)

    pltpu.sync_copy(ids_hbm.at[pl.ds(base, PPT)], ids_v)
    pltpu.sync_copy(pr_hbm.at[pl.ds(base, PPT)], pr_v)
    pltpu.sync_copy(cnt_hbm, cnt_v)

    zero = jnp.zeros((16,), jnp.int32)
    tot = zero
    mybase = zero
    lo_half = LANE < 8
    for i in range(NW // 2):
        v = cnt_v[pl.ds(16 * i, 16)]
        chunk = jnp.where(lo_half, 2 * i, 2 * i + 1)
        tot = tot + v
        mybase = mybase + jnp.where(chunk < wid, v, zero)
    def fold(vec):
        fold_v[pl.ds(0, 16)] = vec
        hi = fold_v[pl.ds(8, 16)]
        return jnp.where(lo_half, vec + hi, zero)
    tot = fold(tot)
    mybase = fold(mybase)

    kept = jnp.minimum(tot, CAP)
    padded = ((kept + (BLK - 1)) >> BLK_SHIFT) << BLK_SHIFT
    padded = jnp.where(lo_half, padded, zero)
    off = plsc.cumsum(padded) - padded
    used_rows = jnp.sum(padded)
    used_blocks = used_rows >> BLK_SHIFT

    base_e = [_lane_scalar(mybase, e) for e in range(E)]
    off_e = [_lane_scalar(off, e) for e in range(E)]
    pad_e = [_lane_scalar(padded, e) for e in range(E)]

    @pl.when(wid == 0)
    def _write_meta():
        for c in range(2):
            lane_g = LANE + 16 * c
            bidx = jnp.clip(lane_g - 1, 0, used_blocks - 1)
            bb = bidx << BLK_SHIFT
            bev = jnp.zeros((16,), jnp.int32)
            for e in range(E):
                hit = (bb >= off_e[e]) & (bb < off_e[e] + pad_e[e])
                bev = jnp.where(hit, e, bev)
            if c == 0:
                bev = jnp.where(lane_g == 0, used_blocks, bev)
            meta_v[pl.ds(16 * c, 16)] = bev
        pltpu.sync_copy(meta_v, meta_hbm)

    run = [jnp.int32(0)] * E
    for j in range(VPT):
        ev = ids_v[pl.ds(16 * j, 16)]
        gv = pr_v[pl.ds(16 * j, 16)]
        dest = jnp.full((16,), DUMP, jnp.int32)
        for e in range(E):
            m = ev == e
            mi = m.astype(jnp.int32)
            incl = plsc.cumsum(mi)
            pos = base_e[e] + run[e] + incl - 1
            ok = m & (pos < CAP)
            dest = jnp.where(ok, off_e[e] + pos, dest)
            gv = jnp.where(m & (pos >= CAP), jnp.float32(0.0), gv)
            run[e] = run[e] + jnp.sum(mi)
        dest_v[pl.ds(16 * j, 16)] = dest
        gate_v[pl.ds(16 * j, 16)] = gv

    pltpu.sync_copy(dest_v, dest_hbm.at[pl.ds(base, PPT)])
    pltpu.sync_copy(gate_v, gate_hbm.at[pl.ds(base, PPT)])

    for jt in range(TPT // 16):
        pidx = 32 * jt + 2 * LANE
        d0_v[pl.ds(16 * jt, 16)] = plsc.load_gather(dest_v, [pidx])
        d1_v[pl.ds(16 * jt, 16)] = plsc.load_gather(dest_v, [pidx + 1])
    rows_dma.wait()
    pltpu.sync_copy(rows_v, xs_hbm.at[d0_v])
    pltpu.sync_copy(rows_v, xs_hbm.at[d1_v])


def _mlp_body(meta_ref, xs_ref, w1_ref, b1_ref, w2_ref, b2_ref, out_ref):
    b = pl.program_id(0)

    @pl.when(b < meta_ref[0])
    def _compute():
        xb = xs_ref[...].astype(jnp.bfloat16)
        w1b = w1_ref[0].astype(jnp.bfloat16)
        h = jnp.dot(xb, w1b, preferred_element_type=jnp.float32)
        h = jnp.maximum(h + b1_ref[0], 0.0).astype(jnp.bfloat16)
        w2b = w2_ref[0].astype(jnp.bfloat16)
        out = jnp.dot(h, w2b, preferred_element_type=jnp.float32)
        out_ref[...] = out + b2_ref[0]


def _combine_body(rows_hbm, dest_hbm, gate_hbm, y_hbm,
                  d0_v, d1_v, gate_v, rows_v, out_v, sem):
    LANE = _lane()
    wid = lax.axis_index("s") * NC + lax.axis_index("c")
    base = wid * PPT
    pltpu.sync_copy(gate_hbm.at[pl.ds(base, PPT)], gate_v)
    half_tok = TPT // 2
    for h in range(2):
        dv = d0_v if h == 0 else d1_v
        pltpu.sync_copy(dest_hbm.at[pl.ds(base + 64 * h, 64)], dv)
        pltpu.async_copy(rows_hbm.at[dv], rows_v, sem).wait()

        def body(tt, _):
            p0 = 64 * h + 2 * tt
            g0 = plsc.load_gather(gate_v, [jnp.full((16,), p0, jnp.int32)])
            g1 = plsc.load_gather(gate_v, [jnp.full((16,), p0 + 1, jnp.int32)])
            r0row = jnp.full((16,), 2 * tt, jnp.int32)
            r1row = jnp.full((16,), 2 * tt + 1, jnp.int32)
            orow = jnp.full((16,), tt, jnp.int32)
            fzero = jnp.zeros((16,), jnp.float32)
            for c in range(D // 16):
                col = LANE + 16 * c
                r0 = plsc.load_gather(rows_v, [r0row, col])
                r1 = plsc.load_gather(rows_v, [r1row, col])
                acc = (jnp.where(g0 != 0.0, g0 * r0, fzero)
                       + jnp.where(g1 != 0.0, g1 * r1, fzero))
                plsc.store_scatter(out_v, [orow, col], acc)
            return 0

        lax.fori_loop(0, half_tok, body, 0)
        pltpu.sync_copy(out_v, y_hbm.at[pl.ds(wid * TPT + half_tok * h,
                                              half_tok)])


@functools.partial(jax.jit, static_argnames=())
def kernel(hidden, W_router, W1, b1, W2, b2):
    x = hidden.reshape(T, D)

    ids, probs, cnt = pl.pallas_call(
        _router_body,
        out_shape=(
            jax.ShapeDtypeStruct((T, TOP_K), jnp.int32),
            jax.ShapeDtypeStruct((T, TOP_K), jnp.float32),
            jax.ShapeDtypeStruct((NW, E), jnp.int32),
        ),
    )(x, W_router)

    ids_flat = ids.reshape(PAIRS)
    pr_flat = probs.reshape(PAIRS)
    cnt_flat = cnt.reshape(NW * E)

    sc_mesh = plsc.VectorSubcoreMesh(core_axis_name="c", subcore_axis_name="s",
                                     num_cores=NC, num_subcores=NS)

    xs, dest, gate, meta = pl.kernel(
        _dispatch_body,
        out_type=(
            jax.ShapeDtypeStruct((XS_ROWS, D), jnp.float32),
            jax.ShapeDtypeStruct((PAIRS,), jnp.int32),
            jax.ShapeDtypeStruct((PAIRS,), jnp.float32),
            jax.ShapeDtypeStruct((32,), jnp.int32),
        ),
        mesh=sc_mesh,
        compiler_params=pltpu.CompilerParams(needs_layout_passes=False),
        scratch_types=[
            pltpu.VMEM((PPT,), jnp.int32),
            pltpu.VMEM((PPT,), jnp.float32),
            pltpu.VMEM((NW * E,), jnp.int32),
            pltpu.VMEM((PPT,), jnp.int32),
            pltpu.VMEM((PPT,), jnp.float32),
            pltpu.VMEM((TPT,), jnp.int32),
            pltpu.VMEM((TPT,), jnp.int32),
            pltpu.VMEM((32,), jnp.int32),
            pltpu.VMEM((24,), jnp.int32),
            pltpu.VMEM((TPT, D), jnp.float32),
            pltpu.SemaphoreType.DMA,
        ],
    )(ids_flat, pr_flat, cnt_flat, x)

    b1r = b1.reshape(E, 1, DFF)
    b2r = b2.reshape(E, 1, D)

    out_rows = pl.pallas_call(
        _mlp_body,
        grid_spec=pltpu.PrefetchScalarGridSpec(
            num_scalar_prefetch=1,
            grid=(NB,),
            in_specs=[
                pl.BlockSpec((BLK, D), lambda b, m: (b, 0)),
                pl.BlockSpec((1, D, DFF), lambda b, m: (m[1 + b], 0, 0)),
                pl.BlockSpec((1, 1, DFF), lambda b, m: (m[1 + b], 0, 0)),
                pl.BlockSpec((1, DFF, D), lambda b, m: (m[1 + b], 0, 0)),
                pl.BlockSpec((1, 1, D), lambda b, m: (m[1 + b], 0, 0)),
            ],
            out_specs=pl.BlockSpec((BLK, D), lambda b, m: (b, 0)),
        ),
        out_shape=jax.ShapeDtypeStruct((XS_ROWS, D), jnp.float32),
    )(meta, xs, W1, b1r, W2, b2r)

    y = pl.kernel(
        _combine_body,
        out_type=jax.ShapeDtypeStruct((T, D), jnp.float32),
        mesh=sc_mesh,
        compiler_params=pltpu.CompilerParams(needs_layout_passes=False),
        scratch_types=[
            pltpu.VMEM((64,), jnp.int32),
            pltpu.VMEM((64,), jnp.int32),
            pltpu.VMEM((PPT,), jnp.float32),
            pltpu.VMEM((64, D), jnp.float32),
            pltpu.VMEM((TPT // 2, D), jnp.float32),
            pltpu.SemaphoreType.DMA,
        ],
    )(out_rows, dest, gate)

    return y.reshape(hidden.shape)
